# baseline (device time: 306725 ns/iter reference)
import jax
import jax.numpy as jnp
import numpy as np
from jax import lax
from jax.experimental import pallas as pl
from jax.experimental.pallas import tpu as pltpu

N_DEV = 32
CAP = 128
E_LOC = 4


def kernel(x, router_W, route_idx, expert_W):
    T, D = x.shape
    _, _, H = expert_W.shape
    W_COLS = D + E_LOC
    my = lax.axis_index("i")

    scores = jnp.dot(x, router_W)
    s01 = jnp.take_along_axis(scores, route_idx, axis=1)
    m = jnp.max(s01, axis=1, keepdims=True)
    g = jnp.exp(s01 - m)
    gn = g / jnp.sum(g, axis=1, keepdims=True)

    a_e = route_idx.reshape(-1)
    a_gate = gn.reshape(-1)
    a_token = jnp.arange(2 * T, dtype=jnp.int32) // 2
    a_off = jnp.mod(a_e // E_LOC - my, N_DEV).astype(jnp.int32)
    a_k = (a_e % E_LOC).astype(jnp.int32)

    order = jnp.argsort(a_off)
    sorted_off = a_off[order]
    counts = jnp.bincount(a_off, length=N_DEV)
    starts = jnp.concatenate([jnp.zeros(1, counts.dtype), jnp.cumsum(counts)[:-1]])
    rank = jnp.arange(2 * T, dtype=jnp.int32) - starts[sorted_off].astype(jnp.int32)
    pos_sorted = sorted_off * CAP + jnp.minimum(rank, CAP - 1)
    pos = jnp.zeros(2 * T, jnp.int32).at[order].set(pos_sorted.astype(jnp.int32))

    inv = jnp.full(N_DEV * CAP, -1, jnp.int32).at[pos].set(
        jnp.arange(2 * T, dtype=jnp.int32))
    valid = inv >= 0
    idxs = jnp.where(valid, inv, 0)
    xg = x[a_token[idxs]] * a_gate[idxs][:, None]
    oh = (a_k[idxs][:, None] == jnp.arange(E_LOC)[None, :]).astype(jnp.float32)
    rows = jnp.concatenate([xg, oh], axis=1)
    rows = jnp.where(valid[:, None], rows, 0.0).astype(jnp.bfloat16)
    send = rows.reshape(N_DEV, CAP, W_COLS)

    def body(send_ref, ew_ref, out_ref, recv_ref, ret_ref, ewbf_ref,
             sx_send, sx_recv, sr_send, sr_recv):
        me = lax.axis_index("i")

        barrier_sem = pltpu.get_barrier_semaphore()
        for o in range(1, N_DEV):
            pl.semaphore_signal(
                barrier_sem, inc=1,
                device_id=(lax.rem(me + o, N_DEV),),
                device_id_type=pl.DeviceIdType.MESH,
            )
        pl.semaphore_wait(barrier_sem, N_DEV - 1)

        for k in range(E_LOC):
            ewbf_ref[k] = ew_ref[k].astype(jnp.bfloat16)

        rdma1 = []
        for o in range(1, N_DEV):
            r = pltpu.make_async_remote_copy(
                src_ref=send_ref.at[o],
                dst_ref=recv_ref.at[o],
                send_sem=sx_send.at[o],
                recv_sem=sx_recv.at[o],
                device_id=(lax.rem(me + o, N_DEV),),
                device_id_type=pl.DeviceIdType.MESH,
            )
            r.start()
            rdma1.append(r)

        def compute(o):
            xo = recv_ref[o, :, 0:D]
            w4 = recv_ref[o, :, D:D + E_LOC].astype(jnp.float32)
            acc = jnp.zeros((CAP, H), jnp.float32)
            for k in range(E_LOC):
                acc += w4[:, k:k + 1] * jnp.dot(
                    xo, ewbf_ref[k], preferred_element_type=jnp.float32)
            ret_ref[o] = acc.astype(jnp.bfloat16)

        recv_ref[0] = send_ref[0]
        compute(0)
        out_ref[0] = ret_ref[0]

        rdma2 = []
        for o in range(1, N_DEV):
            rdma1[o - 1].wait()
            compute(o)
            r = pltpu.make_async_remote_copy(
                src_ref=ret_ref.at[o],
                dst_ref=out_ref.at[o],
                send_sem=sr_send.at[o],
                recv_sem=sr_recv.at[o],
                device_id=(lax.rem(me - o + N_DEV, N_DEV),),
                device_id_type=pl.DeviceIdType.MESH,
            )
            r.start()
            rdma2.append(r)
        for r in rdma2:
            r.wait()

    res = pl.pallas_call(
        body,
        out_shape=jax.ShapeDtypeStruct((N_DEV, CAP, H), jnp.bfloat16),
        in_specs=[pl.BlockSpec(memory_space=pltpu.VMEM)] * 2,
        out_specs=pl.BlockSpec(memory_space=pltpu.VMEM),
        scratch_shapes=[
            pltpu.VMEM((N_DEV, CAP, W_COLS), jnp.bfloat16),
            pltpu.VMEM((N_DEV, CAP, H), jnp.bfloat16),
            pltpu.VMEM((E_LOC, D, H), jnp.bfloat16),
            pltpu.SemaphoreType.DMA((N_DEV,)),
            pltpu.SemaphoreType.DMA((N_DEV,)),
            pltpu.SemaphoreType.DMA((N_DEV,)),
            pltpu.SemaphoreType.DMA((N_DEV,)),
        ],
        compiler_params=pltpu.CompilerParams(collective_id=0),
    )(send, expert_W)

    flat = res.reshape(N_DEV * CAP, H).astype(jnp.float32)
    pos2 = pos.reshape(T, 2)
    return flat[pos2[:, 0]] + flat[pos2[:, 1]]


# device time: 130131 ns/iter; 2.3570x vs baseline; 2.3570x over previous
import jax
import jax.numpy as jnp
from jax import lax
from jax.experimental import pallas as pl
from jax.experimental.pallas import tpu as pltpu

N_DEV = 32
CAP = 128
E_LOC = 4


def kernel(x, router_W, route_idx, expert_W):
    T, D = x.shape
    _, _, H = expert_W.shape
    E = N_DEV * E_LOC
    S = N_DEV * CAP
    W_COLS = D + E_LOC

    def body(x_ref, rw_ref, idx_ref, ew_ref, out_ref,
             send_ref, recv_ref, ret_ref, res_ref, ewbf_ref,
             sx_send, sx_recv, sr_send, sr_recv):
        me = lax.axis_index("i")

        barrier_sem = pltpu.get_barrier_semaphore()
        for o in range(1, N_DEV):
            pl.semaphore_signal(
                barrier_sem, inc=1,
                device_id=(lax.rem(me + o, N_DEV),),
                device_id_type=pl.DeviceIdType.MESH,
            )
        pl.semaphore_wait(barrier_sem, N_DEV - 1)

        scores = jnp.dot(x_ref[:, :], rw_ref[:, :],
                         preferred_element_type=jnp.float32)
        e_iota = lax.broadcasted_iota(jnp.int32, (T, E), 1)
        idx0 = idx_ref[:, 0:1]
        idx1 = idx_ref[:, 1:2]
        s0 = jnp.sum(scores * (e_iota == idx0), axis=1, keepdims=True)
        s1 = jnp.sum(scores * (e_iota == idx1), axis=1, keepdims=True)
        m = jnp.maximum(s0, s1)
        g0 = jnp.exp(s0 - m)
        g1 = jnp.exp(s1 - m)
        gn0 = g0 / (g0 + g1)
        gn1 = g1 / (g0 + g1)

        off0 = jnp.mod(idx0 // E_LOC - me, N_DEV)
        off1 = jnp.mod(idx1 // E_LOC - me, N_DEV)
        k0 = idx0 % E_LOC
        k1 = idx1 % E_LOC

        o_iota = lax.broadcasted_iota(jnp.int32, (T, N_DEV), 1)
        h0 = (o_iota == off0).astype(jnp.float32)
        h1 = (o_iota == off1).astype(jnp.float32)
        c11 = (((1,), (1,)), ((), ()))
        eq00 = lax.dot_general(h0, h0, c11,
                               preferred_element_type=jnp.float32)
        eq11 = lax.dot_general(h1, h1, c11,
                               preferred_element_type=jnp.float32)
        eq10 = lax.dot_general(h1, h0, c11,
                               preferred_element_type=jnp.float32)
        tr = lax.broadcasted_iota(jnp.int32, (T, T), 0)
        tc = lax.broadcasted_iota(jnp.int32, (T, T), 1)
        lower = (tr > tc).astype(jnp.float32)
        rank0 = jnp.sum(eq00 * lower, axis=1, keepdims=True)
        rank1 = (jnp.sum(eq10, axis=1, keepdims=True)
                 + jnp.sum(eq11 * lower, axis=1, keepdims=True))
        pos0 = off0 * CAP + jnp.minimum(rank0.astype(jnp.int32), CAP - 1)
        pos1 = off1 * CAP + jnp.minimum(rank1.astype(jnp.int32), CAP - 1)

        s_iota = lax.broadcasted_iota(jnp.int32, (T, S), 1)
        b0 = (s_iota == pos0).astype(jnp.bfloat16)
        b1 = (s_iota == pos1).astype(jnp.bfloat16)
        k_iota = lax.broadcasted_iota(jnp.int32, (T, E_LOC), 1)
        sel0 = ((k_iota == k0).astype(jnp.float32) * gn0).astype(jnp.bfloat16)
        sel1 = ((k_iota == k1).astype(jnp.float32) * gn1).astype(jnp.bfloat16)
        x_bf = x_ref[:, :].astype(jnp.bfloat16)
        ext0 = jnp.concatenate([x_bf, sel0], axis=1)
        ext1 = jnp.concatenate([x_bf, sel1], axis=1)
        c00 = (((0,), (0,)), ((), ()))
        rows = (lax.dot_general(b0, ext0, c00,
                                preferred_element_type=jnp.float32)
                + lax.dot_general(b1, ext1, c00,
                                  preferred_element_type=jnp.float32))
        send_ref[:, :, :] = rows.astype(jnp.bfloat16).reshape(N_DEV, CAP, W_COLS)

        for k in range(E_LOC):
            ewbf_ref[k] = ew_ref[k].astype(jnp.bfloat16)

        rdma1 = []
        for o in range(1, N_DEV):
            r = pltpu.make_async_remote_copy(
                src_ref=send_ref.at[o],
                dst_ref=recv_ref.at[o],
                send_sem=sx_send.at[o],
                recv_sem=sx_recv.at[o],
                device_id=(lax.rem(me + o, N_DEV),),
                device_id_type=pl.DeviceIdType.MESH,
            )
            r.start()
            rdma1.append(r)

        def compute(o):
            xo = recv_ref[o, :, 0:D]
            w4 = recv_ref[o, :, D:D + E_LOC].astype(jnp.float32)
            acc = jnp.zeros((CAP, H), jnp.float32)
            for k in range(E_LOC):
                acc += w4[:, k:k + 1] * jnp.dot(
                    xo, ewbf_ref[k], preferred_element_type=jnp.float32)
            ret_ref[o] = acc.astype(jnp.bfloat16)

        recv_ref[0] = send_ref[0]
        compute(0)
        res_ref[0] = ret_ref[0]

        rdma2 = []
        for o in range(1, N_DEV):
            rdma1[o - 1].wait()
            compute(o)
            r = pltpu.make_async_remote_copy(
                src_ref=ret_ref.at[o],
                dst_ref=res_ref.at[o],
                send_sem=sr_send.at[o],
                recv_sem=sr_recv.at[o],
                device_id=(lax.rem(me - o + N_DEV, N_DEV),),
                device_id_type=pl.DeviceIdType.MESH,
            )
            r.start()
            rdma2.append(r)
        for r in rdma2:
            r.wait()

        res_flat = res_ref[:, :, :].reshape(S, H)
        out_ref[:, :] = jnp.dot((b0 + b1).astype(jnp.bfloat16), res_flat,
                                preferred_element_type=jnp.float32)

    return pl.pallas_call(
        body,
        out_shape=jax.ShapeDtypeStruct((T, H), jnp.float32),
        in_specs=[pl.BlockSpec(memory_space=pltpu.VMEM)] * 4,
        out_specs=pl.BlockSpec(memory_space=pltpu.VMEM),
        scratch_shapes=[
            pltpu.VMEM((N_DEV, CAP, W_COLS), jnp.bfloat16),
            pltpu.VMEM((N_DEV, CAP, W_COLS), jnp.bfloat16),
            pltpu.VMEM((N_DEV, CAP, H), jnp.bfloat16),
            pltpu.VMEM((N_DEV, CAP, H), jnp.bfloat16),
            pltpu.VMEM((E_LOC, D, H), jnp.bfloat16),
            pltpu.SemaphoreType.DMA((N_DEV,)),
            pltpu.SemaphoreType.DMA((N_DEV,)),
            pltpu.SemaphoreType.DMA((N_DEV,)),
            pltpu.SemaphoreType.DMA((N_DEV,)),
        ],
        compiler_params=pltpu.CompilerParams(
            collective_id=0, vmem_limit_bytes=100 * 1024 * 1024),
    )(x, router_W, route_idx, expert_W)


# device time: 97297 ns/iter; 3.1525x vs baseline; 1.3375x over previous
import jax
import jax.numpy as jnp
from jax import lax
from jax.experimental import pallas as pl
from jax.experimental.pallas import tpu as pltpu

N_DEV = 32
CAP = 112
E_LOC = 4


def kernel(x, router_W, route_idx, expert_W):
    T, D = x.shape
    _, _, H = expert_W.shape
    E = N_DEV * E_LOC
    S = N_DEV * CAP
    ROWS = CAP + E_LOC

    def body(x_ref, rw_ref, idx_ref, ew_ref, out_ref,
             send_ref, recv_ref, ret_ref, res_ref, ewbf_ref,
             sx_send, sx_recv, sr_send, sr_recv):
        me = lax.axis_index("i")

        barrier_sem = pltpu.get_barrier_semaphore()
        for o in range(1, N_DEV):
            pl.semaphore_signal(
                barrier_sem, inc=1,
                device_id=(lax.rem(me + o, N_DEV),),
                device_id_type=pl.DeviceIdType.MESH,
            )
        pl.semaphore_wait(barrier_sem, N_DEV - 1)

        scores = jnp.dot(x_ref[:, :], rw_ref[:, :],
                         preferred_element_type=jnp.float32)
        e_iota = lax.broadcasted_iota(jnp.int32, (T, E), 1)
        idx0 = idx_ref[:, 0:1]
        idx1 = idx_ref[:, 1:2]
        s0 = jnp.sum(scores * (e_iota == idx0), axis=1, keepdims=True)
        s1 = jnp.sum(scores * (e_iota == idx1), axis=1, keepdims=True)
        m = jnp.maximum(s0, s1)
        g0 = jnp.exp(s0 - m)
        g1 = jnp.exp(s1 - m)
        gn0 = g0 / (g0 + g1)
        gn1 = g1 / (g0 + g1)

        off0 = jnp.mod(idx0 // E_LOC - me, N_DEV)
        off1 = jnp.mod(idx1 // E_LOC - me, N_DEV)
        k0 = idx0 % E_LOC
        k1 = idx1 % E_LOC

        o_iota = lax.broadcasted_iota(jnp.int32, (T, N_DEV), 1)
        h0 = (o_iota == off0).astype(jnp.float32)
        h1 = (o_iota == off1).astype(jnp.float32)
        c11 = (((1,), (1,)), ((), ()))
        eq00 = lax.dot_general(h0, h0, c11,
                               preferred_element_type=jnp.float32)
        eq11 = lax.dot_general(h1, h1, c11,
                               preferred_element_type=jnp.float32)
        eq10 = lax.dot_general(h1, h0, c11,
                               preferred_element_type=jnp.float32)
        tr = lax.broadcasted_iota(jnp.int32, (T, T), 0)
        tc = lax.broadcasted_iota(jnp.int32, (T, T), 1)
        lower = (tr > tc).astype(jnp.float32)
        rank0 = jnp.sum(eq00 * lower, axis=1, keepdims=True)
        rank1 = (jnp.sum(eq10, axis=1, keepdims=True)
                 + jnp.sum(eq11 * lower, axis=1, keepdims=True))
        pos0 = off0 * CAP + jnp.minimum(rank0.astype(jnp.int32), CAP - 1)
        pos1 = off1 * CAP + jnp.minimum(rank1.astype(jnp.int32), CAP - 1)

        s_iota = lax.broadcasted_iota(jnp.int32, (T, S), 1)
        b0 = (s_iota == pos0).astype(jnp.bfloat16)
        b1 = (s_iota == pos1).astype(jnp.bfloat16)
        bsum = b0 + b1
        k_iota = lax.broadcasted_iota(jnp.int32, (T, E_LOC), 1)
        sel0 = ((k_iota == k0).astype(jnp.float32) * gn0).astype(jnp.bfloat16)
        sel1 = ((k_iota == k1).astype(jnp.float32) * gn1).astype(jnp.bfloat16)
        x_bf = x_ref[:, :].astype(jnp.bfloat16)
        c00 = (((0,), (0,)), ((), ()))

        for k in range(E_LOC):
            ewbf_ref[k] = ew_ref[k].astype(jnp.bfloat16)

        def make_rdma1(o):
            return pltpu.make_async_remote_copy(
                src_ref=send_ref.at[o],
                dst_ref=recv_ref.at[o],
                send_sem=sx_send.at[o],
                recv_sem=sx_recv.at[o],
                device_id=(lax.rem(me + o, N_DEV),),
                device_id_type=pl.DeviceIdType.MESH,
            )

        rdma1 = {}
        bounds = [0, 8, 16, 24, N_DEV]
        for lo, hi in zip(bounds[:-1], bounds[1:]):
            sl = slice(lo * CAP, hi * CAP)
            rows_x = lax.dot_general(bsum[:, sl], x_bf, c00,
                                     preferred_element_type=jnp.float32)
            rows_selT = (lax.dot_general(sel0, b0[:, sl], c00,
                                         preferred_element_type=jnp.float32)
                         + lax.dot_general(sel1, b1[:, sl], c00,
                                           preferred_element_type=jnp.float32))
            send_ref[lo:hi, 0:CAP, :] = rows_x.astype(jnp.bfloat16).reshape(
                hi - lo, CAP, D)
            selT_bf = rows_selT.astype(jnp.bfloat16)
            for j, o in enumerate(range(lo, hi)):
                send_ref[o, CAP:ROWS, 0:CAP] = selT_bf[:, j * CAP:(j + 1) * CAP]
            for o in range(max(lo, 1), hi):
                rdma1[o] = make_rdma1(o)
                rdma1[o].start()

        eye_r = lax.broadcasted_iota(jnp.int32, (CAP, CAP), 0)
        eye_c = lax.broadcasted_iota(jnp.int32, (CAP, CAP), 1)
        eye = (eye_r == eye_c).astype(jnp.bfloat16)
        c11b = (((1,), (1,)), ((), ()))

        def compute(o):
            xo = recv_ref[o, 0:CAP, :]
            w4t = recv_ref[o, CAP:ROWS, 0:CAP]
            w4 = lax.dot_general(eye, w4t, c11b,
                                 preferred_element_type=jnp.float32)
            acc = jnp.zeros((CAP, H), jnp.float32)
            for k in range(E_LOC):
                acc += w4[:, k:k + 1] * jnp.dot(
                    xo, ewbf_ref[k], preferred_element_type=jnp.float32)
            ret_ref[o] = acc.astype(jnp.bfloat16)

        recv_ref[0] = send_ref[0]
        compute(0)
        res_ref[0] = ret_ref[0]

        rdma2 = []
        for o in range(1, N_DEV):
            rdma1[o].wait()
            compute(o)
            r = pltpu.make_async_remote_copy(
                src_ref=ret_ref.at[o],
                dst_ref=res_ref.at[o],
                send_sem=sr_send.at[o],
                recv_sem=sr_recv.at[o],
                device_id=(lax.rem(me - o + N_DEV, N_DEV),),
                device_id_type=pl.DeviceIdType.MESH,
            )
            r.start()
            rdma2.append(r)

        out_ref[:, :] = jnp.dot(bsum[:, 0:CAP], res_ref[0],
                                preferred_element_type=jnp.float32)
        for o in range(1, N_DEV):
            rdma2[o - 1].wait()
            out_ref[:, :] += jnp.dot(bsum[:, o * CAP:(o + 1) * CAP],
                                     res_ref[o],
                                     preferred_element_type=jnp.float32)

    return pl.pallas_call(
        body,
        out_shape=jax.ShapeDtypeStruct((T, H), jnp.float32),
        in_specs=[pl.BlockSpec(memory_space=pltpu.VMEM)] * 4,
        out_specs=pl.BlockSpec(memory_space=pltpu.VMEM),
        scratch_shapes=[
            pltpu.VMEM((N_DEV, ROWS, D), jnp.bfloat16),
            pltpu.VMEM((N_DEV, ROWS, D), jnp.bfloat16),
            pltpu.VMEM((N_DEV, CAP, H), jnp.bfloat16),
            pltpu.VMEM((N_DEV, CAP, H), jnp.bfloat16),
            pltpu.VMEM((E_LOC, D, H), jnp.bfloat16),
            pltpu.SemaphoreType.DMA((N_DEV,)),
            pltpu.SemaphoreType.DMA((N_DEV,)),
            pltpu.SemaphoreType.DMA((N_DEV,)),
            pltpu.SemaphoreType.DMA((N_DEV,)),
        ],
        compiler_params=pltpu.CompilerParams(
            collective_id=0, vmem_limit_bytes=100 * 1024 * 1024),
    )(x, router_W, route_idx, expert_W)
